# baseline (speedup 1.0000x reference)
import functools

import numpy as np
import jax
import jax.numpy as jnp
from jax import lax
from jax.experimental import pallas as pl
from jax.experimental.pallas import tpu as pltpu
from jax.experimental.pallas import tpu_sc as plsc

_B, _L = 16, 512
_L2 = _L + 2
_LP = 528
_LI = 640
_LO = 640
_OUTW = 1056
_ROWS = _B * 4


_U32 = np.uint32


def _tf2x32(k0, k1, x0, x1):
    ks2 = _U32(k0 ^ k1 ^ _U32(0x1BD11BDA))

    def rotl(v, d):
        return ((v << _U32(d)) | (v >> _U32(32 - d))).astype(_U32)

    def rounds(v0, v1, rots):
        for rr in rots:
            v0 = (v0 + v1).astype(_U32)
            v1 = rotl(v1, rr)
            v1 = (v0 ^ v1).astype(_U32)
        return v0, v1

    r0 = (13, 15, 26, 6)
    r1 = (17, 29, 16, 24)
    v0 = (x0.astype(_U32) + k0).astype(_U32)
    v1 = (x1.astype(_U32) + k1).astype(_U32)
    v0, v1 = rounds(v0, v1, r0)
    v0 = (v0 + k1).astype(_U32); v1 = (v1 + ks2 + _U32(1)).astype(_U32)
    v0, v1 = rounds(v0, v1, r1)
    v0 = (v0 + ks2).astype(_U32); v1 = (v1 + k0 + _U32(2)).astype(_U32)
    v0, v1 = rounds(v0, v1, r0)
    v0 = (v0 + k0).astype(_U32); v1 = (v1 + k1 + _U32(3)).astype(_U32)
    v0, v1 = rounds(v0, v1, r1)
    v0 = (v0 + k1).astype(_U32); v1 = (v1 + ks2 + _U32(4)).astype(_U32)
    v0, v1 = rounds(v0, v1, r0)
    v0 = (v0 + ks2).astype(_U32); v1 = (v1 + k0 + _U32(5)).astype(_U32)
    return v0, v1


def _np_bits(kd, m):
    i = np.arange(m, dtype=_U32)
    b1, b2 = _tf2x32(kd[0], kd[1], np.zeros(m, _U32), i)
    return (b1 ^ b2).astype(_U32)


def _np_split(kd, n):
    i = np.arange(n, dtype=_U32)
    b1, b2 = _tf2x32(kd[0], kd[1], np.zeros(n, _U32), i)
    return [(b1[j], b2[j]) for j in range(n)]


def _np_fold_in(kd, data):
    o0, o1 = _tf2x32(kd[0], kd[1], np.zeros(1, _U32),
                     np.array([data & 0xFFFFFFFF], _U32))
    return (o0[0], o1[0])


def _np_uniform(kd, m):
    bits = _np_bits(kd, m)
    fb = ((bits >> _U32(9)) | _U32(0x3F800000)).astype(_U32)
    fl = fb.view(np.float32) - np.float32(1.0)
    return np.maximum(np.float32(0.0), fl)


def _np_randint4(kd, m):
    kv = _np_split(kd, 2)[1]
    return (_np_bits(kv, m) & _U32(3)).astype(np.float32)


def _build_consts():
    kd = (_U32(0), _U32(1234))
    keys = _np_split(kd, 4)
    subu = np.zeros((4, _B, _L), np.float32)
    subd = np.zeros((4, _B, _L), np.float32)
    delu = np.zeros((4, _B, _L), np.float32)
    insu = np.full((4, _B, _LI), 9.0, np.float32)
    insv = np.zeros((4, _B, _LI), np.float32)
    for c in range(4):
        ks, kdel, ki = _np_split(keys[c], 3)
        kd_, kp_ = _np_split(ks, 2)
        subd[c] = _np_randint4(kd_, _B * _L).reshape(_B, _L)
        subu[c] = _np_uniform(kp_, _B * _L).reshape(_B, _L)
        delu[c] = _np_uniform(kdel, _B * _L).reshape(_B, _L)
        for b in range(_B):
            kb = _np_fold_in(ki, b)
            kp2, kv2 = _np_split(kb, 2)
            insu[c, b, :_L2] = _np_uniform(kp2, _L2)
            insv[c, b, :_L2] = _np_randint4(kv2, _L2)
    insv += 1.0

    def flat(a):
        return a.transpose(1, 0, 2).reshape(_ROWS, -1)

    blk = np.concatenate([flat(subu), flat(subd), flat(delu),
                          flat(insu), flat(insv)], axis=1)
    return np.ascontiguousarray(blk.reshape(-1))


_CONST = _build_consts()
_O_SUBU = 0
_O_SUBD = _O_SUBU + _L
_O_DELU = _O_SUBD + _L
_O_INSU = _O_DELU + _L
_O_INSV = _O_INSU + _LI
_BLK = _O_INSV + _LI


@functools.lru_cache(maxsize=1)
def _get_sc_fn():
    mesh = plsc.VectorSubcoreMesh(core_axis_name="c", subcore_axis_name="s")
    nc = mesh.num_cores
    nw = nc * mesh.num_subcores
    rows_per_w = _ROWS // nw

    @functools.partial(
        pl.kernel,
        out_type=jax.ShapeDtypeStruct((_ROWS * _LO,), jnp.float32),
        mesh=mesh,
        compiler_params=pltpu.CompilerParams(needs_layout_passes=False,
                                             use_tc_tiling_on_sc=True),
        scratch_types=[
            [pltpu.VMEM((_L,), jnp.float32)] * rows_per_w,
            [pltpu.VMEM((_BLK,), jnp.float32)] * rows_per_w,
            [pltpu.VMEM((_OUTW,), jnp.float32)] * rows_per_w,
            [pltpu.VMEM((_LP,), jnp.float32)] * rows_per_w,
            pltpu.VMEM((16,), jnp.float32),
            pltpu.VMEM((16,), jnp.float32),
            pltpu.VMEM((16,), jnp.float32),
            pltpu.SemaphoreType.DMA,
        ],
    )
    def _sc_channel(seg, const, sub_e, del_e, ins_e,
                    out, x_v, c_v, o_v, y_v, e1_v, e2_v, e3_v, sem):
        wid = lax.axis_index("s") * nc + lax.axis_index("c")
        it16 = lax.iota(jnp.int32, 16)
        zero16 = jnp.zeros((16,), jnp.float32)

        herr = [pltpu.async_copy(sub_e, e1_v.at[pl.ds(0, 1)], sem),
                pltpu.async_copy(del_e, e2_v.at[pl.ds(0, 1)], sem),
                pltpu.async_copy(ins_e, e3_v.at[pl.ds(0, 1)], sem)]
        rows = [wid * rows_per_w + k for k in range(rows_per_w)]
        hin = []
        for k, r in enumerate(rows):
            hin.append([
                pltpu.async_copy(seg.at[r // 4], x_v[k], sem),
                pltpu.async_copy(const.at[pl.ds(r * _BLK, _BLK)],
                                 c_v[k], sem)])
        for h in herr:
            h.wait()
        sub_t = e1_v[pl.ds(0, 16)][0]
        del_t = e2_v[pl.ds(0, 16)][0]
        ins_t = e3_v[pl.ds(0, 16)][0]

        hout = []
        for k in range(rows_per_w):
            for h in hin[k]:
                h.wait()
        nvec = [jnp.zeros((16,), jnp.int32) for _ in range(rows_per_w)]
        for v in range(_L // 16):
            o = 16 * v
            for k in range(rows_per_w):
                blk = c_v[k]
                xx = x_v[k][pl.ds(o, 16)]
                su = blk[pl.ds(_O_SUBU + o, 16)]
                sd = blk[pl.ds(_O_SUBD + o, 16)]
                du = blk[pl.ds(_O_DELU + o, 16)]
                s = xx + jnp.where(su < sub_t, sd, 0.0)
                x1p = jnp.where(s < 4.0, s + 1.0, s - 3.0)
                keep = du >= del_t
                ki32 = keep.astype(jnp.int32)
                excl = plsc.cumsum(ki32) - ki32
                plsc.store_scatter(y_v[k], [excl + nvec[k]], x1p, mask=keep)
                nvec[k] = nvec[k] + plsc.all_reduce_population_count(keep)
        cvec = [jnp.zeros((16,), jnp.int32) for _ in range(rows_per_w)]
        for v in range(_LP // 16):
            o = 16 * v
            ivec = it16 + o
            for k in range(rows_per_w):
                blk = c_v[k]
                inb = ivec < nvec[k]
                act = (blk[pl.ds(_O_INSU + o, 16)] < ins_t) & inb
                ai = act.astype(jnp.int32)
                excl = plsc.cumsum(ai) - ai
                cl = excl + cvec[k]
                pos = ivec + cl
                yv = jnp.where(inb, y_v[k][pl.ds(o, 16)], 0.0)
                plsc.store_scatter(o_v[k], [pos], yv)
                vals = plsc.load_gather(blk.at[pl.ds(_O_INSV, _LI)], [cl])
                plsc.store_scatter(o_v[k], [pos + 1], vals, mask=act)
                cvec[k] = cvec[k] + plsc.all_reduce_population_count(act)
        for k in range(rows_per_w):
            r = rows[k]
            hout.append(pltpu.async_copy(o_v[k].at[pl.ds(0, _LO)],
                                         out.at[pl.ds(r * _LO, _LO)], sem))
        for h in hout:
            h.wait()

    return _sc_channel


def kernel(segment_en, sub_error, del_error, ins_error):
    flat = _get_sc_fn()(segment_en.astype(jnp.float32), _CONST,
                        sub_error.astype(jnp.float32),
                        del_error.astype(jnp.float32),
                        ins_error.astype(jnp.float32))
    return flat.reshape(_B, 4, _LO)[:, :, :_L2]


# speedup vs baseline: 1.1627x; 1.1627x over previous
import functools

import numpy as np
import jax
import jax.numpy as jnp
from jax import lax
from jax.experimental import pallas as pl
from jax.experimental.pallas import tpu as pltpu
from jax.experimental.pallas import tpu_sc as plsc

_B, _L = 16, 512
_L2 = _L + 2
_LP = 528
_LI = 640
_LO = 640
_OUTW = 1056
_ROWS = _B * 4


_U32 = np.uint32


def _tf2x32(k0, k1, x0, x1):
    ks2 = _U32(k0 ^ k1 ^ _U32(0x1BD11BDA))

    def rotl(v, d):
        return ((v << _U32(d)) | (v >> _U32(32 - d))).astype(_U32)

    def rounds(v0, v1, rots):
        for rr in rots:
            v0 = (v0 + v1).astype(_U32)
            v1 = rotl(v1, rr)
            v1 = (v0 ^ v1).astype(_U32)
        return v0, v1

    r0 = (13, 15, 26, 6)
    r1 = (17, 29, 16, 24)
    v0 = (x0.astype(_U32) + k0).astype(_U32)
    v1 = (x1.astype(_U32) + k1).astype(_U32)
    v0, v1 = rounds(v0, v1, r0)
    v0 = (v0 + k1).astype(_U32); v1 = (v1 + ks2 + _U32(1)).astype(_U32)
    v0, v1 = rounds(v0, v1, r1)
    v0 = (v0 + ks2).astype(_U32); v1 = (v1 + k0 + _U32(2)).astype(_U32)
    v0, v1 = rounds(v0, v1, r0)
    v0 = (v0 + k0).astype(_U32); v1 = (v1 + k1 + _U32(3)).astype(_U32)
    v0, v1 = rounds(v0, v1, r1)
    v0 = (v0 + k1).astype(_U32); v1 = (v1 + ks2 + _U32(4)).astype(_U32)
    v0, v1 = rounds(v0, v1, r0)
    v0 = (v0 + ks2).astype(_U32); v1 = (v1 + k0 + _U32(5)).astype(_U32)
    return v0, v1


def _np_bits(kd, m):
    i = np.arange(m, dtype=_U32)
    b1, b2 = _tf2x32(kd[0], kd[1], np.zeros(m, _U32), i)
    return (b1 ^ b2).astype(_U32)


def _np_split(kd, n):
    i = np.arange(n, dtype=_U32)
    b1, b2 = _tf2x32(kd[0], kd[1], np.zeros(n, _U32), i)
    return [(b1[j], b2[j]) for j in range(n)]


def _np_fold_in(kd, data):
    o0, o1 = _tf2x32(kd[0], kd[1], np.zeros(1, _U32),
                     np.array([data & 0xFFFFFFFF], _U32))
    return (o0[0], o1[0])


def _np_uniform(kd, m):
    bits = _np_bits(kd, m)
    fb = ((bits >> _U32(9)) | _U32(0x3F800000)).astype(_U32)
    fl = fb.view(np.float32) - np.float32(1.0)
    return np.maximum(np.float32(0.0), fl)


def _np_randint4(kd, m):
    kv = _np_split(kd, 2)[1]
    return (_np_bits(kv, m) & _U32(3)).astype(np.float32)


def _build_consts():
    kd = (_U32(0), _U32(1234))
    keys = _np_split(kd, 4)
    subu = np.zeros((4, _B, _L), np.float32)
    subd = np.zeros((4, _B, _L), np.float32)
    delu = np.zeros((4, _B, _L), np.float32)
    insu = np.full((4, _B, _LI), 9.0, np.float32)
    insv = np.zeros((4, _B, _LI), np.float32)
    for c in range(4):
        ks, kdel, ki = _np_split(keys[c], 3)
        kd_, kp_ = _np_split(ks, 2)
        subd[c] = _np_randint4(kd_, _B * _L).reshape(_B, _L)
        subu[c] = _np_uniform(kp_, _B * _L).reshape(_B, _L)
        delu[c] = _np_uniform(kdel, _B * _L).reshape(_B, _L)
        for b in range(_B):
            kb = _np_fold_in(ki, b)
            kp2, kv2 = _np_split(kb, 2)
            insu[c, b, :_L2] = _np_uniform(kp2, _L2)
            insv[c, b, :_L2] = _np_randint4(kv2, _L2)
    insv += 1.0

    def flat(a):
        return a.transpose(1, 0, 2).reshape(_ROWS, -1)

    blk = np.concatenate([flat(subu), flat(subd), flat(delu),
                          flat(insu), flat(insv)], axis=1)
    return np.ascontiguousarray(blk.reshape(-1))


_CONST = _build_consts()
_O_SUBU = 0
_O_SUBD = _O_SUBU + _L
_O_DELU = _O_SUBD + _L
_O_INSU = _O_DELU + _L
_O_INSV = _O_INSU + _LI
_BLK = _O_INSV + _LI


@functools.lru_cache(maxsize=1)
def _get_sc_fn():
    mesh = plsc.VectorSubcoreMesh(core_axis_name="c", subcore_axis_name="s")
    nc = mesh.num_cores
    nw = nc * mesh.num_subcores
    rows_per_w = _ROWS // nw

    @functools.partial(
        pl.kernel,
        out_type=jax.ShapeDtypeStruct((_ROWS * _LO,), jnp.float32),
        mesh=mesh,
        compiler_params=pltpu.CompilerParams(needs_layout_passes=False,
                                             use_tc_tiling_on_sc=True),
        scratch_types=[
            [pltpu.VMEM((_L,), jnp.float32)] * rows_per_w,
            [pltpu.VMEM((_BLK,), jnp.float32)] * rows_per_w,
            [pltpu.VMEM((_OUTW,), jnp.float32)] * rows_per_w,
            [pltpu.VMEM((_LP,), jnp.float32)] * rows_per_w,
            pltpu.VMEM((16,), jnp.float32),
            pltpu.VMEM((16,), jnp.float32),
            pltpu.VMEM((16,), jnp.float32),
            pltpu.SemaphoreType.DMA,
        ],
    )
    def _sc_channel(seg, const, sub_e, del_e, ins_e,
                    out, x_v, c_v, o_v, y_v, e1_v, e2_v, e3_v, sem):
        wid = lax.axis_index("s") * nc + lax.axis_index("c")
        it16 = lax.iota(jnp.int32, 16)
        zero16 = jnp.zeros((16,), jnp.float32)

        herr = [pltpu.async_copy(sub_e, e1_v.at[pl.ds(0, 1)], sem),
                pltpu.async_copy(del_e, e2_v.at[pl.ds(0, 1)], sem),
                pltpu.async_copy(ins_e, e3_v.at[pl.ds(0, 1)], sem)]
        rows = [wid * rows_per_w + k for k in range(rows_per_w)]
        hin = []
        for k, r in enumerate(rows):
            hin.append([
                pltpu.async_copy(seg.at[r // 4], x_v[k], sem),
                pltpu.async_copy(const.at[pl.ds(r * _BLK, _BLK)],
                                 c_v[k], sem)])
        for h in herr:
            h.wait()
        sub_t = e1_v[pl.ds(0, 16)][0]
        del_t = e2_v[pl.ds(0, 16)][0]
        ins_t = e3_v[pl.ds(0, 16)][0]

        hout = []
        for k in range(rows_per_w):
            for h in hin[k]:
                h.wait()
        def body_a(v, nv):
            o = v * 16
            out_nv = []
            for k in range(rows_per_w):
                blk = c_v[k]
                xx = x_v[k][pl.ds(o, 16)]
                su = blk[pl.ds(_O_SUBU + o, 16)]
                sd = blk[pl.ds(_O_SUBD + o, 16)]
                du = blk[pl.ds(_O_DELU + o, 16)]
                s = xx + jnp.where(su < sub_t, sd, 0.0)
                x1p = jnp.where(s < 4.0, s + 1.0, s - 3.0)
                keep = du >= del_t
                ki32 = keep.astype(jnp.int32)
                excl = plsc.cumsum(ki32) - ki32
                plsc.store_scatter(y_v[k], [excl + nv[k]], x1p, mask=keep)
                out_nv.append(nv[k] + plsc.all_reduce_population_count(keep))
            return tuple(out_nv)

        nvec = lax.fori_loop(0, _L // 16, body_a,
                             tuple(jnp.zeros((16,), jnp.int32)
                                   for _ in range(rows_per_w)))

        def body_b(v, cv):
            o = v * 16
            ivec = it16 + o
            out_cv = []
            for k in range(rows_per_w):
                blk = c_v[k]
                inb = ivec < nvec[k]
                act = (blk[pl.ds(_O_INSU + o, 16)] < ins_t) & inb
                ai = act.astype(jnp.int32)
                excl = plsc.cumsum(ai) - ai
                cl = excl + cv[k]
                pos = ivec + cl
                yv = jnp.where(inb, y_v[k][pl.ds(o, 16)], 0.0)
                plsc.store_scatter(o_v[k], [pos], yv)
                vals = plsc.load_gather(blk.at[pl.ds(_O_INSV, _LI)], [cl])
                plsc.store_scatter(o_v[k], [pos + 1], vals, mask=act)
                out_cv.append(cv[k] + plsc.all_reduce_population_count(act))
            return tuple(out_cv)

        lax.fori_loop(0, _LP // 16, body_b,
                      tuple(jnp.zeros((16,), jnp.int32)
                            for _ in range(rows_per_w)))
        for k in range(rows_per_w):
            r = rows[k]
            hout.append(pltpu.async_copy(o_v[k].at[pl.ds(0, _LO)],
                                         out.at[pl.ds(r * _LO, _LO)], sem))
        for h in hout:
            h.wait()

    return _sc_channel


def kernel(segment_en, sub_error, del_error, ins_error):
    flat = _get_sc_fn()(segment_en.astype(jnp.float32), _CONST,
                        sub_error.astype(jnp.float32),
                        del_error.astype(jnp.float32),
                        ins_error.astype(jnp.float32))
    return flat.reshape(_B, 4, _LO)[:, :, :_L2]


# speedup vs baseline: 1.2439x; 1.0698x over previous
import functools

import numpy as np
import jax
import jax.numpy as jnp
from jax import lax
from jax.experimental import pallas as pl
from jax.experimental.pallas import tpu as pltpu
from jax.experimental.pallas import tpu_sc as plsc

_B, _L = 16, 512
_L2 = _L + 2
_LP = 528
_LI = 640
_LO = 640
_OUTW = 1056
_ROWS = _B * 4


_U32 = np.uint32


def _tf2x32(k0, k1, x0, x1):
    ks2 = _U32(k0 ^ k1 ^ _U32(0x1BD11BDA))

    def rotl(v, d):
        return ((v << _U32(d)) | (v >> _U32(32 - d))).astype(_U32)

    def rounds(v0, v1, rots):
        for rr in rots:
            v0 = (v0 + v1).astype(_U32)
            v1 = rotl(v1, rr)
            v1 = (v0 ^ v1).astype(_U32)
        return v0, v1

    r0 = (13, 15, 26, 6)
    r1 = (17, 29, 16, 24)
    v0 = (x0.astype(_U32) + k0).astype(_U32)
    v1 = (x1.astype(_U32) + k1).astype(_U32)
    v0, v1 = rounds(v0, v1, r0)
    v0 = (v0 + k1).astype(_U32); v1 = (v1 + ks2 + _U32(1)).astype(_U32)
    v0, v1 = rounds(v0, v1, r1)
    v0 = (v0 + ks2).astype(_U32); v1 = (v1 + k0 + _U32(2)).astype(_U32)
    v0, v1 = rounds(v0, v1, r0)
    v0 = (v0 + k0).astype(_U32); v1 = (v1 + k1 + _U32(3)).astype(_U32)
    v0, v1 = rounds(v0, v1, r1)
    v0 = (v0 + k1).astype(_U32); v1 = (v1 + ks2 + _U32(4)).astype(_U32)
    v0, v1 = rounds(v0, v1, r0)
    v0 = (v0 + ks2).astype(_U32); v1 = (v1 + k0 + _U32(5)).astype(_U32)
    return v0, v1


def _np_bits(kd, m):
    i = np.arange(m, dtype=_U32)
    b1, b2 = _tf2x32(kd[0], kd[1], np.zeros(m, _U32), i)
    return (b1 ^ b2).astype(_U32)


def _np_split(kd, n):
    i = np.arange(n, dtype=_U32)
    b1, b2 = _tf2x32(kd[0], kd[1], np.zeros(n, _U32), i)
    return [(b1[j], b2[j]) for j in range(n)]


def _np_fold_in(kd, data):
    o0, o1 = _tf2x32(kd[0], kd[1], np.zeros(1, _U32),
                     np.array([data & 0xFFFFFFFF], _U32))
    return (o0[0], o1[0])


def _np_uniform(kd, m):
    bits = _np_bits(kd, m)
    fb = ((bits >> _U32(9)) | _U32(0x3F800000)).astype(_U32)
    fl = fb.view(np.float32) - np.float32(1.0)
    return np.maximum(np.float32(0.0), fl)


def _np_randint4(kd, m):
    kv = _np_split(kd, 2)[1]
    return (_np_bits(kv, m) & _U32(3)).astype(np.float32)


def _build_consts():
    kd = (_U32(0), _U32(1234))
    keys = _np_split(kd, 4)
    subu = np.zeros((4, _B, _L), np.float32)
    subd = np.zeros((4, _B, _L), np.float32)
    delu = np.zeros((4, _B, _L), np.float32)
    insu = np.full((4, _B, _LI), 9.0, np.float32)
    insv = np.zeros((4, _B, _LI), np.float32)
    for c in range(4):
        ks, kdel, ki = _np_split(keys[c], 3)
        kd_, kp_ = _np_split(ks, 2)
        subd[c] = _np_randint4(kd_, _B * _L).reshape(_B, _L)
        subu[c] = _np_uniform(kp_, _B * _L).reshape(_B, _L)
        delu[c] = _np_uniform(kdel, _B * _L).reshape(_B, _L)
        for b in range(_B):
            kb = _np_fold_in(ki, b)
            kp2, kv2 = _np_split(kb, 2)
            insu[c, b, :_L2] = _np_uniform(kp2, _L2)
            insv[c, b, :_L2] = _np_randint4(kv2, _L2)
    insv += 1.0

    def flat(a):
        return a.transpose(1, 0, 2).reshape(_ROWS, -1)

    blk = np.concatenate([flat(subu), flat(subd), flat(delu),
                          flat(insu), flat(insv)], axis=1)
    return np.ascontiguousarray(blk.reshape(-1))


_CONST = _build_consts()
_O_SUBU = 0
_O_SUBD = _O_SUBU + _L
_O_DELU = _O_SUBD + _L
_O_INSU = _O_DELU + _L
_O_INSV = _O_INSU + _LI
_BLK = _O_INSV + _LI


@functools.lru_cache(maxsize=1)
def _get_sc_fn():
    mesh = plsc.VectorSubcoreMesh(core_axis_name="c", subcore_axis_name="s")
    nc = mesh.num_cores
    nw = nc * mesh.num_subcores
    rows_per_w = _ROWS // nw

    @functools.partial(
        pl.kernel,
        out_type=jax.ShapeDtypeStruct((_B, 4, _LO), jnp.float32),
        mesh=mesh,
        compiler_params=pltpu.CompilerParams(needs_layout_passes=False,
                                             use_tc_tiling_on_sc=True),
        scratch_types=[
            [pltpu.VMEM((_L,), jnp.float32)] * rows_per_w,
            [pltpu.VMEM((_BLK,), jnp.float32)] * rows_per_w,
            [pltpu.VMEM((_OUTW,), jnp.float32)] * rows_per_w,
            [pltpu.VMEM((_LP,), jnp.float32)] * rows_per_w,
            pltpu.VMEM((16,), jnp.float32),
            pltpu.VMEM((16,), jnp.float32),
            pltpu.VMEM((16,), jnp.float32),
            pltpu.SemaphoreType.DMA,
        ],
    )
    def _sc_channel(seg, const, sub_e, del_e, ins_e,
                    out, x_v, c_v, o_v, y_v, e1_v, e2_v, e3_v, sem):
        wid = lax.axis_index("s") * nc + lax.axis_index("c")
        it16 = lax.iota(jnp.int32, 16)
        zero16 = jnp.zeros((16,), jnp.float32)

        herr = [pltpu.async_copy(sub_e, e1_v.at[pl.ds(0, 1)], sem),
                pltpu.async_copy(del_e, e2_v.at[pl.ds(0, 1)], sem),
                pltpu.async_copy(ins_e, e3_v.at[pl.ds(0, 1)], sem)]
        rows = [wid * rows_per_w + k for k in range(rows_per_w)]
        hin = []
        for k, r in enumerate(rows):
            hin.append([
                pltpu.async_copy(seg.at[r // 4], x_v[k], sem),
                pltpu.async_copy(const.at[pl.ds(r * _BLK, _BLK)],
                                 c_v[k], sem)])
        for h in herr:
            h.wait()
        sub_t = e1_v[pl.ds(0, 16)][0]
        del_t = e2_v[pl.ds(0, 16)][0]
        ins_t = e3_v[pl.ds(0, 16)][0]

        hout = []
        for k in range(rows_per_w):
            for h in hin[k]:
                h.wait()
        def body_a(v, nv):
            o = v * 16
            out_nv = []
            for k in range(rows_per_w):
                blk = c_v[k]
                xx = x_v[k][pl.ds(o, 16)]
                su = blk[pl.ds(_O_SUBU + o, 16)]
                sd = blk[pl.ds(_O_SUBD + o, 16)]
                du = blk[pl.ds(_O_DELU + o, 16)]
                s = xx + jnp.where(su < sub_t, sd, 0.0)
                x1p = jnp.where(s < 4.0, s + 1.0, s - 3.0)
                keep = du >= del_t
                ki32 = keep.astype(jnp.int32)
                excl = plsc.cumsum(ki32) - ki32
                plsc.store_scatter(y_v[k], [excl + nv[k]], x1p, mask=keep)
                out_nv.append(nv[k] + plsc.all_reduce_population_count(keep))
            return tuple(out_nv)

        nvec = lax.fori_loop(0, _L // 16, body_a,
                             tuple(jnp.zeros((16,), jnp.int32)
                                   for _ in range(rows_per_w)))

        def body_b(v, cv):
            o = v * 16
            ivec = it16 + o
            out_cv = []
            for k in range(rows_per_w):
                blk = c_v[k]
                inb = ivec < nvec[k]
                act = (blk[pl.ds(_O_INSU + o, 16)] < ins_t) & inb
                ai = act.astype(jnp.int32)
                excl = plsc.cumsum(ai) - ai
                cl = excl + cv[k]
                pos = ivec + cl
                yv = jnp.where(inb, y_v[k][pl.ds(o, 16)], 0.0)
                plsc.store_scatter(o_v[k], [pos], yv)
                vals = plsc.load_gather(blk.at[pl.ds(_O_INSV, _LI)], [cl])
                plsc.store_scatter(o_v[k], [pos + 1], vals, mask=act)
                out_cv.append(cv[k] + plsc.all_reduce_population_count(act))
            return tuple(out_cv)

        lax.fori_loop(0, _LP // 16, body_b,
                      tuple(jnp.zeros((16,), jnp.int32)
                            for _ in range(rows_per_w)))
        for k in range(rows_per_w):
            r = rows[k]
            hout.append(pltpu.async_copy(o_v[k].at[pl.ds(0, _LO)],
                                         out.at[r // 4, r % 4], sem))
        for h in hout:
            h.wait()

    return _sc_channel


def kernel(segment_en, sub_error, del_error, ins_error):
    padded = _get_sc_fn()(segment_en.astype(jnp.float32), _CONST,
                          sub_error.astype(jnp.float32),
                          del_error.astype(jnp.float32),
                          ins_error.astype(jnp.float32))
    return padded[:, :, :_L2]


# speedup vs baseline: 1.2467x; 1.0022x over previous
import functools

import numpy as np
import jax
import jax.numpy as jnp
from jax import lax
from jax.experimental import pallas as pl
from jax.experimental.pallas import tpu as pltpu
from jax.experimental.pallas import tpu_sc as plsc

_B, _L = 16, 512
_L2 = _L + 2
_LP = 528
_LI = 640
_LO = 640
_OUTW = 1056
_ROWS = _B * 4


_U32 = np.uint32


def _tf2x32(k0, k1, x0, x1):
    ks2 = _U32(k0 ^ k1 ^ _U32(0x1BD11BDA))

    def rotl(v, d):
        return ((v << _U32(d)) | (v >> _U32(32 - d))).astype(_U32)

    def rounds(v0, v1, rots):
        for rr in rots:
            v0 = (v0 + v1).astype(_U32)
            v1 = rotl(v1, rr)
            v1 = (v0 ^ v1).astype(_U32)
        return v0, v1

    r0 = (13, 15, 26, 6)
    r1 = (17, 29, 16, 24)
    v0 = (x0.astype(_U32) + k0).astype(_U32)
    v1 = (x1.astype(_U32) + k1).astype(_U32)
    v0, v1 = rounds(v0, v1, r0)
    v0 = (v0 + k1).astype(_U32); v1 = (v1 + ks2 + _U32(1)).astype(_U32)
    v0, v1 = rounds(v0, v1, r1)
    v0 = (v0 + ks2).astype(_U32); v1 = (v1 + k0 + _U32(2)).astype(_U32)
    v0, v1 = rounds(v0, v1, r0)
    v0 = (v0 + k0).astype(_U32); v1 = (v1 + k1 + _U32(3)).astype(_U32)
    v0, v1 = rounds(v0, v1, r1)
    v0 = (v0 + k1).astype(_U32); v1 = (v1 + ks2 + _U32(4)).astype(_U32)
    v0, v1 = rounds(v0, v1, r0)
    v0 = (v0 + ks2).astype(_U32); v1 = (v1 + k0 + _U32(5)).astype(_U32)
    return v0, v1


def _np_bits(kd, m):
    i = np.arange(m, dtype=_U32)
    b1, b2 = _tf2x32(kd[0], kd[1], np.zeros(m, _U32), i)
    return (b1 ^ b2).astype(_U32)


def _np_split(kd, n):
    i = np.arange(n, dtype=_U32)
    b1, b2 = _tf2x32(kd[0], kd[1], np.zeros(n, _U32), i)
    return [(b1[j], b2[j]) for j in range(n)]


def _np_fold_in(kd, data):
    o0, o1 = _tf2x32(kd[0], kd[1], np.zeros(1, _U32),
                     np.array([data & 0xFFFFFFFF], _U32))
    return (o0[0], o1[0])


def _np_uniform(kd, m):
    bits = _np_bits(kd, m)
    fb = ((bits >> _U32(9)) | _U32(0x3F800000)).astype(_U32)
    fl = fb.view(np.float32) - np.float32(1.0)
    return np.maximum(np.float32(0.0), fl)


def _np_randint4(kd, m):
    kv = _np_split(kd, 2)[1]
    return (_np_bits(kv, m) & _U32(3)).astype(np.float32)


def _build_consts():
    kd = (_U32(0), _U32(1234))
    keys = _np_split(kd, 4)
    subu = np.zeros((4, _B, _L), np.float32)
    subd = np.zeros((4, _B, _L), np.float32)
    delu = np.zeros((4, _B, _L), np.float32)
    insu = np.full((4, _B, _LI), 9.0, np.float32)
    insv = np.zeros((4, _B, _LI), np.float32)
    for c in range(4):
        ks, kdel, ki = _np_split(keys[c], 3)
        kd_, kp_ = _np_split(ks, 2)
        subd[c] = _np_randint4(kd_, _B * _L).reshape(_B, _L)
        subu[c] = _np_uniform(kp_, _B * _L).reshape(_B, _L)
        delu[c] = _np_uniform(kdel, _B * _L).reshape(_B, _L)
        for b in range(_B):
            kb = _np_fold_in(ki, b)
            kp2, kv2 = _np_split(kb, 2)
            insu[c, b, :_L2] = _np_uniform(kp2, _L2)
            insv[c, b, :_L2] = _np_randint4(kv2, _L2)
    insv += 1.0

    def flat(a):
        return a.transpose(1, 0, 2).reshape(_ROWS, -1)

    blk = np.concatenate([flat(subu), flat(subd), flat(delu),
                          flat(insu), flat(insv)], axis=1)
    return np.ascontiguousarray(blk.reshape(-1))


_CONST = _build_consts()
_O_SUBU = 0
_O_SUBD = _O_SUBU + _L
_O_DELU = _O_SUBD + _L
_O_INSU = _O_DELU + _L
_O_INSV = _O_INSU + _LI
_BLK = _O_INSV + _LI


@functools.lru_cache(maxsize=1)
def _get_sc_fn():
    mesh = plsc.VectorSubcoreMesh(core_axis_name="c", subcore_axis_name="s")
    nc = mesh.num_cores
    rows_per_w = _ROWS // (nc * mesh.num_subcores)

    @functools.partial(
        pl.kernel,
        out_type=jax.ShapeDtypeStruct((_B, 4, _LO), jnp.float32),
        mesh=mesh,
        compiler_params=pltpu.CompilerParams(needs_layout_passes=False,
                                             use_tc_tiling_on_sc=True),
        scratch_types=[
            [pltpu.VMEM((_L,), jnp.float32)] * rows_per_w,
            [pltpu.VMEM((_BLK,), jnp.float32)] * rows_per_w,
            [pltpu.VMEM((_OUTW,), jnp.float32)] * rows_per_w,
            [pltpu.VMEM((_LP,), jnp.float32)] * rows_per_w,
            pltpu.VMEM((16,), jnp.float32),
            pltpu.VMEM((16,), jnp.float32),
            pltpu.VMEM((16,), jnp.float32),
            pltpu.SemaphoreType.DMA,
        ],
    )
    def _sc_channel(seg, const, sub_e, del_e, ins_e,
                    out, x_v, c_v, o_v, y_v, e1_v, e2_v, e3_v, sem):
        wid = lax.axis_index("s") * nc + lax.axis_index("c")
        it16 = lax.iota(jnp.int32, 16)

        herr = [pltpu.async_copy(sub_e, e1_v.at[pl.ds(0, 1)], sem),
                pltpu.async_copy(del_e, e2_v.at[pl.ds(0, 1)], sem),
                pltpu.async_copy(ins_e, e3_v.at[pl.ds(0, 1)], sem)]
        rows = [wid * rows_per_w + k for k in range(rows_per_w)]
        hin = []
        for k, r in enumerate(rows):
            hin.append([
                pltpu.async_copy(seg.at[r // 4], x_v[k], sem),
                pltpu.async_copy(const.at[pl.ds(r * _BLK, _BLK)],
                                 c_v[k], sem)])
        for h in herr:
            h.wait()
        sub_t = e1_v[pl.ds(0, 16)][0]
        del_t = e2_v[pl.ds(0, 16)][0]
        ins_t = e3_v[pl.ds(0, 16)][0]

        hout = []
        for k in range(rows_per_w):
            for h in hin[k]:
                h.wait()
        def body_a(v, nv):
            o = v * 16
            out_nv = []
            for k in range(rows_per_w):
                blk = c_v[k]
                xx = x_v[k][pl.ds(o, 16)]
                su = blk[pl.ds(_O_SUBU + o, 16)]
                sd = blk[pl.ds(_O_SUBD + o, 16)]
                du = blk[pl.ds(_O_DELU + o, 16)]
                s = xx + jnp.where(su < sub_t, sd, 0.0)
                x1p = jnp.where(s < 4.0, s + 1.0, s - 3.0)
                keep = du >= del_t
                ki32 = keep.astype(jnp.int32)
                excl = plsc.cumsum(ki32) - ki32
                plsc.store_scatter(y_v[k], [excl + nv[k]], x1p, mask=keep)
                out_nv.append(nv[k] + plsc.all_reduce_population_count(keep))
            return tuple(out_nv)

        nvec = lax.fori_loop(0, _L // 16, body_a,
                             tuple(jnp.zeros((16,), jnp.int32)
                                   for _ in range(rows_per_w)))

        def body_b(v, cv):
            o = v * 16
            ivec = it16 + o
            out_cv = []
            for k in range(rows_per_w):
                blk = c_v[k]
                inb = ivec < nvec[k]
                act = (blk[pl.ds(_O_INSU + o, 16)] < ins_t) & inb
                ai = act.astype(jnp.int32)
                excl = plsc.cumsum(ai) - ai
                cl = excl + cv[k]
                pos = ivec + cl
                yv = jnp.where(inb, y_v[k][pl.ds(o, 16)], 0.0)
                plsc.store_scatter(o_v[k], [pos], yv)
                vals = plsc.load_gather(blk.at[pl.ds(_O_INSV, _LI)], [cl])
                plsc.store_scatter(o_v[k], [pos + 1], vals, mask=act)
                out_cv.append(cv[k] + plsc.all_reduce_population_count(act))
            return tuple(out_cv)

        lax.fori_loop(0, _LP // 16, body_b,
                      tuple(jnp.zeros((16,), jnp.int32)
                            for _ in range(rows_per_w)))
        for k in range(rows_per_w):
            r = rows[k]
            hout.append(pltpu.async_copy(o_v[k].at[pl.ds(0, _LO)],
                                         out.at[r // 4, r % 4], sem))
        for h in hout:
            h.wait()

    return _sc_channel


def kernel(segment_en, sub_error, del_error, ins_error):
    padded = _get_sc_fn()(segment_en.astype(jnp.float32), _CONST,
                          sub_error.astype(jnp.float32),
                          del_error.astype(jnp.float32),
                          ins_error.astype(jnp.float32))
    return padded[:, :, :_L2]
